# baseline (speedup 1.0000x reference)
import functools

import jax
import jax.numpy as jnp
from jax import lax
from jax.experimental import pallas as pl
from jax.experimental.pallas import tpu as pltpu
from jax.experimental.pallas import tpu_sc as plsc

B = 16384
D = 32
NC = 2
NS = 16
NW = NC * NS
B_PER_W = B // NW
CHUNK = 128
NCHUNK = B_PER_W // CHUNK


def _gather_body(ut, it, ui, ii, u_out, i_out, idx_u, idx_i, rows_u, rows_i, sem):
    wid = lax.axis_index("s") * NC + lax.axis_index("c")
    base = wid * B_PER_W
    pltpu.sync_copy(ui.at[pl.ds(base, B_PER_W)], idx_u)
    pltpu.sync_copy(ii.at[pl.ds(base, B_PER_W)], idx_i)
    copies = []
    for c in range(NCHUNK):
        sl = pl.ds(c * CHUNK, CHUNK)
        copies.append(pltpu.async_copy(ut.at[idx_u.at[sl]], rows_u.at[sl], sem))
        copies.append(pltpu.async_copy(it.at[idx_i.at[sl]], rows_i.at[sl], sem))
    for cp in copies:
        cp.wait()
    pltpu.sync_copy(rows_u, u_out.at[pl.ds(base, B_PER_W)])
    pltpu.sync_copy(rows_i, i_out.at[pl.ds(base, B_PER_W)])


@jax.jit
def _sc_gather(user_table, item_table, user_idx, item_idx):
    mesh = plsc.VectorSubcoreMesh(core_axis_name="c", subcore_axis_name="s")
    fn = pl.kernel(
        _gather_body,
        mesh=mesh,
        compiler_params=pltpu.CompilerParams(use_tc_tiling_on_sc=False),
        out_type=[
            jax.ShapeDtypeStruct((B, D), jnp.float32),
            jax.ShapeDtypeStruct((B, D), jnp.float32),
        ],
        scratch_types=[
            pltpu.VMEM((B_PER_W,), jnp.int32),
            pltpu.VMEM((B_PER_W,), jnp.int32),
            pltpu.VMEM((B_PER_W, D), jnp.float32),
            pltpu.VMEM((B_PER_W, D), jnp.float32),
            pltpu.SemaphoreType.DMA,
        ],
    )
    return fn(user_table, item_table, user_idx, item_idx)


def _mlp_body(u_ref, i_ref, w1u_ref, w1i_ref, b1_ref, w2_ref, b2_ref,
              w3t_ref, b3_ref, o_ref):
    h = u_ref[...] @ w1u_ref[...] + i_ref[...] @ w1i_ref[...] + b1_ref[...]
    h = jnp.maximum(h, 0.0)
    h = jnp.maximum(h @ w2_ref[...] + b2_ref[...], 0.0)
    z = jnp.sum(h * w3t_ref[...], axis=1, keepdims=True) + b3_ref[...]
    o_ref[...] = jax.nn.sigmoid(z) * 4.0 + 1.0


@jax.jit
def _tc_mlp(u_emb, i_emb, W1u, W1i, b1, W2, b2, W3t, b3):
    bm = 2048
    grid = (B // bm,)
    full = lambda shape: pl.BlockSpec(shape, lambda ib: (0, 0))
    return pl.pallas_call(
        _mlp_body,
        grid=grid,
        in_specs=[
            pl.BlockSpec((bm, D), lambda ib: (ib, 0)),
            pl.BlockSpec((bm, D), lambda ib: (ib, 0)),
            full((D, 64)),
            full((D, 64)),
            full((1, 64)),
            full((64, 32)),
            full((1, 32)),
            full((1, 32)),
            full((1, 1)),
        ],
        out_specs=pl.BlockSpec((bm, 1), lambda ib: (ib, 0)),
        out_shape=jax.ShapeDtypeStruct((B, 1), jnp.float32),
    )(u_emb, i_emb, W1u, W1i, b1, W2, b2, W3t, b3)


def kernel(user_idx, item_idx, user_table, item_table, W1, b1, W2, b2, W3, b3):
    user_idx = user_idx.astype(jnp.int32)
    item_idx = item_idx.astype(jnp.int32)
    u_emb, i_emb = _sc_gather(user_table, item_table, user_idx, item_idx)
    return _tc_mlp(
        u_emb, i_emb,
        W1[:D], W1[D:],
        b1.reshape(1, 64),
        W2, b2.reshape(1, 32),
        W3.reshape(1, 32), b3.reshape(1, 1),
    )
